# Initial kernel scaffold; baseline (speedup 1.0000x reference)
import jax
import jax.numpy as jnp
from jax.experimental import pallas as pl


def kernel(data_orig, road_map, cluster_table):
    raise NotImplementedError("write your pallas kernel here")


# speedup vs baseline: 10.8195x; 10.8195x over previous
import functools

import jax
import jax.numpy as jnp
from jax import lax
from jax.experimental import pallas as pl
from jax.experimental.pallas import tpu as pltpu
from jax.experimental.pallas import tpu_sc as plsc

ROUTEID_NUM = 100000
PAD_ID = ROUTEID_NUM + 1
CLUSTER_NUM = 10000
EMBED_SIZE = 64

_info = plsc.get_sparse_core_info()
_NC, _NS = _info.num_cores, _info.num_subcores
_NW = _NC * _NS

_N = 4096 * 200
_BPW = _N // _NW
_C = 512
_NCHUNK = _BPW // _C


def _sc_lookup(flat_idx, road_map2, table2):
    mesh = plsc.VectorSubcoreMesh(core_axis_name="c", subcore_axis_name="s")

    @functools.partial(
        pl.kernel,
        mesh=mesh,
        compiler_params=pltpu.CompilerParams(use_tc_tiling_on_sc=False),
        out_type=jax.ShapeDtypeStruct((_N, EMBED_SIZE), jnp.float32),
        scratch_types=[
            pltpu.VMEM((_BPW,), jnp.int32),
            pltpu.VMEM((_BPW,), jnp.int32),
            pltpu.VMEM((_C, EMBED_SIZE), jnp.float32),
            pltpu.VMEM((_C, EMBED_SIZE), jnp.float32),
            pltpu.SemaphoreType.DMA,
            pltpu.SemaphoreType.DMA,
            pltpu.SemaphoreType.DMA,
        ],
    )
    def k(idx_hbm, rmap_hbm, tbl_hbm, out_hbm,
          idx_v, cid_v, rows_a, rows_b, sem_i, sem_a, sem_b):
        wid = lax.axis_index("s") * _NC + lax.axis_index("c")
        base = wid * _BPW

        pltpu.sync_copy(idx_hbm.at[pl.ds(base, _BPW)], idx_v)
        pltpu.async_copy(rmap_hbm.at[idx_v], cid_v, sem_i).wait()

        bufs = (rows_a, rows_b)
        sems = (sem_a, sem_b)

        def gather(c, buf, sem):
            return pltpu.async_copy(
                tbl_hbm.at[cid_v.at[pl.ds(c * _C, _C)]], buf, sem)

        gather(0, bufs[0], sems[0])

        def step(c, _):
            par = lax.rem(c, 2)

            def handle(b):
                @pl.when(par == b)
                def _():
                    nxt = c + 1

                    @pl.when(nxt < _NCHUNK)
                    def _():
                        gather(nxt, bufs[1 - b], sems[1 - b])

                    pltpu.make_async_copy(
                        tbl_hbm.at[cid_v.at[pl.ds(0, _C)]],
                        bufs[b], sems[b]).wait()
                    pltpu.sync_copy(bufs[b],
                                    out_hbm.at[pl.ds(base + c * _C, _C)])

            handle(0)
            handle(1)
            return 0

        lax.fori_loop(0, _NCHUNK, step, 0)

    return k(flat_idx, road_map2, table2)


def kernel(data_orig, road_map, cluster_table):
    flat = data_orig.reshape(-1)
    table2 = jnp.concatenate(
        [cluster_table, jnp.zeros((1, EMBED_SIZE), jnp.float32)], axis=0)
    road_map2 = road_map.at[PAD_ID].set(CLUSTER_NUM)
    out = _sc_lookup(flat, road_map2, table2)
    return out.reshape(data_orig.shape[0], data_orig.shape[1], EMBED_SIZE)


# speedup vs baseline: 14.2537x; 1.3174x over previous
import functools

import jax
import jax.numpy as jnp
from jax import lax
from jax.experimental import pallas as pl
from jax.experimental.pallas import tpu as pltpu
from jax.experimental.pallas import tpu_sc as plsc

ROUTEID_NUM = 100000
PAD_ID = ROUTEID_NUM + 1
CLUSTER_NUM = 10000
EMBED_SIZE = 64

_info = plsc.get_sparse_core_info()
_NC, _NS = _info.num_cores, _info.num_subcores
_NW = _NC * _NS

_N = 4096 * 200
_BPW = _N // _NW
_RMAP_PAD = 100096
_C = 320
_NCHUNK = _BPW // _C

_mesh = plsc.VectorSubcoreMesh(core_axis_name="c", subcore_axis_name="s")


def _wid():
    return lax.axis_index("s") * _NC + lax.axis_index("c")


@functools.partial(
    pl.kernel,
    mesh=_mesh,
    compiler_params=pltpu.CompilerParams(use_tc_tiling_on_sc=False),
    out_type=jax.ShapeDtypeStruct((_N,), jnp.int32),
    scratch_types=[
        pltpu.VMEM((_BPW,), jnp.int32),
        pltpu.VMEM((_BPW,), jnp.int32),
        pltpu.SemaphoreType.DMA,
    ],
)
def _level1(idx_hbm, rmap_hbm, cid_hbm, idx_v, cid_v, sem):
    base = _wid() * _BPW
    pltpu.sync_copy(idx_hbm.at[pl.ds(base, _BPW)], idx_v)
    pltpu.async_copy(rmap_hbm.at[idx_v], cid_v, sem).wait()
    pltpu.sync_copy(cid_v, cid_hbm.at[pl.ds(base, _BPW)])


@functools.partial(
    pl.kernel,
    mesh=_mesh,
    out_type=jax.ShapeDtypeStruct((_N, 2 * EMBED_SIZE), jnp.float32),
    scratch_types=[
        pltpu.VMEM((_BPW,), jnp.int32),
        pltpu.VMEM((_C, 2 * EMBED_SIZE), jnp.float32),
        pltpu.VMEM((_C, 2 * EMBED_SIZE), jnp.float32),
        pltpu.SemaphoreType.DMA,
        pltpu.SemaphoreType.DMA,
    ],
)
def _level2(cid_hbm, tbl_hbm, out_hbm, cid_v, rows_a, rows_b, sem_a, sem_b):
    base = _wid() * _BPW
    pltpu.sync_copy(cid_hbm.at[pl.ds(base, _BPW)], cid_v)

    bufs = (rows_a, rows_b)
    sems = (sem_a, sem_b)

    def gather(c, buf, sem):
        return pltpu.async_copy(tbl_hbm.at[cid_v.at[pl.ds(c * _C, _C)]],
                                buf, sem)

    gather(0, bufs[0], sems[0])

    def step(c, _):
        par = lax.rem(c, 2)

        def handle(b):
            @pl.when(par == b)
            def _():
                nxt = c + 1

                @pl.when(nxt < _NCHUNK)
                def _():
                    gather(nxt, bufs[1 - b], sems[1 - b])

                pltpu.make_async_copy(
                    tbl_hbm.at[cid_v.at[pl.ds(0, _C)]],
                    bufs[b], sems[b]).wait()
                pltpu.sync_copy(bufs[b],
                                out_hbm.at[pl.ds(base + c * _C, _C)])

        handle(0)
        handle(1)
        return 0

    lax.fori_loop(0, _NCHUNK, step, 0)


def kernel(data_orig, road_map, cluster_table):
    flat = data_orig.reshape(-1)
    road_map2 = jnp.pad(road_map.at[PAD_ID].set(CLUSTER_NUM),
                        (0, _RMAP_PAD - (ROUTEID_NUM + 2)))
    table2 = jnp.pad(cluster_table,
                     ((0, 1), (0, EMBED_SIZE)))
    cid = _level1(flat, road_map2)
    out = _level2(cid, table2)
    out = out[:, :EMBED_SIZE]
    return out.reshape(data_orig.shape[0], data_orig.shape[1], EMBED_SIZE)
